# baseline (speedup 1.0000x reference)
import jax
import jax.numpy as jnp
from jax.experimental import pallas as pl
from jax.experimental.pallas import tpu as pltpu

_ROWS, _COLS = 32, 32
_LR0 = 0.5
_SIGMA0 = max(_ROWS, _COLS) / 2.0
_B, _D = 256, 512
_M = _ROWS * _COLS

_T = 256
_NT = _M // _T


def _dot3(ah, al, bh, bl, dims):
    acc = jax.lax.dot_general(ah, bh, dims, preferred_element_type=jnp.float32)
    acc += jax.lax.dot_general(ah, bl, dims, preferred_element_type=jnp.float32)
    acc += jax.lax.dot_general(al, bh, dims, preferred_element_type=jnp.float32)
    return acc


def _som_body(e_ref, t_ref, batch_ref, w_ref, out_ref,
              min_ref, idx_ref, wsave_ref, bh_ref, bl_ref):
    i = pl.program_id(0)

    @pl.when(i == 0)
    def _split_batch():
        b = batch_ref[:]
        bh = b.astype(jnp.bfloat16)
        bh_ref[:] = bh
        bl_ref[:] = (b - bh.astype(jnp.float32)).astype(jnp.bfloat16)

    @pl.when(i < _NT)
    def _bmu_phase():
        w = w_ref[:]
        wsave_ref[pl.ds(i * _T, _T), :] = w
        wh = w.astype(jnp.bfloat16)
        wl = (w - wh.astype(jnp.float32)).astype(jnp.bfloat16)
        wn = jnp.sum(w * w, axis=1, keepdims=True)
        dims = (((1,), (1,)), ((), ()))
        st = wn - 2.0 * _dot3(wh, wl, bh_ref[:], bl_ref[:], dims)
        tmin = jnp.min(st, axis=0, keepdims=True)
        midx = _T * i + jax.lax.broadcasted_iota(jnp.int32, (_T, _B), 0)
        tidx = jnp.min(jnp.where(st == tmin, midx, _M), axis=0, keepdims=True)

        @pl.when(i == 0)
        def _():
            min_ref[0:1, :] = tmin
            idx_ref[0:1, :] = tidx

        @pl.when(i > 0)
        def _():
            better = tmin < min_ref[0:1, :]
            min_ref[0:1, :] = jnp.where(better, tmin, min_ref[0:1, :])
            idx_ref[0:1, :] = jnp.where(better, tidx, idx_ref[0:1, :])

    @pl.when(i >= _NT)
    def _update_phase():
        ratio = -(e_ref[0].astype(jnp.float32) / t_ref[0].astype(jnp.float32))
        bmu = idx_ref[0:1, :]
        br = (bmu // _COLS).astype(jnp.float32)
        bc = (bmu % _COLS).astype(jnp.float32)
        m2 = _T * (i - _NT) + jax.lax.broadcasted_iota(jnp.int32, (_T, _B), 0)
        mr = (m2 // _COLS).astype(jnp.float32)
        mc = (m2 % _COLS).astype(jnp.float32)
        nd2 = (mr - br) ** 2 + (mc - bc) ** 2
        coef = jnp.exp(jnp.full((1, _B), -2.0 * ratio)) * (-0.5 / (_SIGMA0 * _SIGMA0))
        h = jnp.exp(ratio + nd2 * coef)
        hsum = jnp.sum(h, axis=1, keepdims=True)
        hx = jax.lax.dot_general(
            h, batch_ref[:], (((1,), (0,)), ((), ())),
            preferred_element_type=jnp.float32,
        )
        w = wsave_ref[pl.ds((i - _NT) * _T, _T), :]
        out_ref[:] = w + (_LR0 / _B) * (hx - hsum * w)


def kernel(batch, weights, epoch, total_epochs):
    e = jnp.asarray(epoch, jnp.int32).reshape(1)
    t = jnp.asarray(total_epochs, jnp.int32).reshape(1)
    return pl.pallas_call(
        _som_body,
        grid=(2 * _NT,),
        out_shape=jax.ShapeDtypeStruct((_M, _D), jnp.float32),
        in_specs=[
            pl.BlockSpec(memory_space=pltpu.SMEM),
            pl.BlockSpec(memory_space=pltpu.SMEM),
            pl.BlockSpec((_B, _D), lambda i: (0, 0)),
            pl.BlockSpec((_T, _D), lambda i: (jnp.where(i < _NT, i, _NT - 1), 0)),
        ],
        out_specs=pl.BlockSpec(
            (_T, _D), lambda i: (jnp.where(i < _NT, 0, i - _NT), 0)
        ),
        scratch_shapes=[
            pltpu.VMEM((8, _B), jnp.float32),
            pltpu.VMEM((8, _B), jnp.int32),
            pltpu.VMEM((_M, _D), jnp.float32),
            pltpu.VMEM((_B, _D), jnp.bfloat16),
            pltpu.VMEM((_B, _D), jnp.bfloat16),
        ],
    )(e, t, batch, weights)


# speedup vs baseline: 1.1896x; 1.1896x over previous
import jax
import jax.numpy as jnp
from jax.experimental import pallas as pl
from jax.experimental.pallas import tpu as pltpu

_ROWS, _COLS = 32, 32
_LR0 = 0.5
_SIGMA0 = max(_ROWS, _COLS) / 2.0
_B, _D = 256, 512
_M = _ROWS * _COLS


def _som_body(e_ref, t_ref, batch_ref, w_ref, out_ref):
    b = batch_ref[:]
    bh = b.astype(jnp.bfloat16)
    bl = (b - bh.astype(jnp.float32)).astype(jnp.bfloat16)
    w = w_ref[:]
    wh = w.astype(jnp.bfloat16)
    wl = (w - wh.astype(jnp.float32)).astype(jnp.bfloat16)

    dims = (((1,), (1,)), ((), ()))
    dot = jax.lax.dot_general(wh, bh, dims, preferred_element_type=jnp.float32)
    dot += jax.lax.dot_general(wh, bl, dims, preferred_element_type=jnp.float32)
    dot += jax.lax.dot_general(wl, bh, dims, preferred_element_type=jnp.float32)
    wn = jnp.sum(w * w, axis=1, keepdims=True)
    st = wn - 2.0 * dot
    tmin = jnp.min(st, axis=0, keepdims=True)
    midx = jax.lax.broadcasted_iota(jnp.int32, (_M, _B), 0)
    bmu = jnp.min(jnp.where(st == tmin, midx, _M), axis=0, keepdims=True)

    ratio = -(e_ref[0].astype(jnp.float32) / t_ref[0].astype(jnp.float32))
    br = (bmu // _COLS).astype(jnp.float32)
    bc = (bmu % _COLS).astype(jnp.float32)
    mr = (midx // _COLS).astype(jnp.float32)
    mc = (midx % _COLS).astype(jnp.float32)
    nd2 = (mr - br) ** 2 + (mc - bc) ** 2
    coef = jnp.exp(jnp.full((1, _B), -2.0 * ratio)) * (-0.5 / (_SIGMA0 * _SIGMA0))
    h = jnp.exp(ratio + nd2 * coef)

    hsum = jnp.sum(h, axis=1, keepdims=True)
    hx = jax.lax.dot_general(
        h, b, (((1,), (0,)), ((), ())),
        preferred_element_type=jnp.float32,
    )
    out_ref[:] = w + (_LR0 / _B) * (hx - hsum * w)


def kernel(batch, weights, epoch, total_epochs):
    e = jnp.asarray(epoch, jnp.int32).reshape(1)
    t = jnp.asarray(total_epochs, jnp.int32).reshape(1)
    return pl.pallas_call(
        _som_body,
        out_shape=jax.ShapeDtypeStruct((_M, _D), jnp.float32),
        in_specs=[
            pl.BlockSpec(memory_space=pltpu.SMEM),
            pl.BlockSpec(memory_space=pltpu.SMEM),
            pl.BlockSpec(memory_space=pltpu.VMEM),
            pl.BlockSpec(memory_space=pltpu.VMEM),
        ],
        out_specs=pl.BlockSpec(memory_space=pltpu.VMEM),
    )(e, t, batch, weights)
